# Initial kernel scaffold; baseline (speedup 1.0000x reference)
import jax
import jax.numpy as jnp
from jax.experimental import pallas as pl


def kernel(x, dca_W, dca_b, cen_W, cen_b, coh_w, coh_b, ee_W, ee_b, vlm_enc_W, vlm_enc_b, vlm_dec_W, vlm_dec_b):
    raise NotImplementedError("write your pallas kernel here")


# speedup vs baseline: 2.8867x; 2.8867x over previous
import functools

import jax
import jax.numpy as jnp
from jax.experimental import pallas as pl
from jax.experimental.pallas import tpu as pltpu

_SPARSITY = 0.8
_THRESHOLD = 0.85
_BISECT_ITERS = 22
_TILE = 512


_INV_SQRT2 = 0.7071067811865476


def _gelu(v):
    return 0.5 * v * (1.0 + jax.lax.erf(v * _INV_SQRT2))


def _kwta_mask(h, k):
    ah = jnp.abs(h)
    mx = jnp.max(ah, axis=-1, keepdims=True)
    lo = jnp.zeros_like(mx)
    hi = mx * (1.0 + 2.0 ** -12) + 1e-30
    kf = jnp.float32(k)

    def body(_, carry):
        lo, hi = carry
        mid = 0.5 * (lo + hi)
        cnt = jnp.sum((ah >= mid).astype(jnp.float32), axis=-1, keepdims=True)
        pred = cnt >= kf
        return jnp.where(pred, mid, lo), jnp.where(pred, hi, mid)

    lo, hi = jax.lax.fori_loop(0, _BISECT_ITERS, body, (lo, hi))
    return ah >= lo


def _conf(logits):
    m = jnp.max(logits, axis=-1, keepdims=True)
    se = jnp.sum(jnp.exp(logits - m), axis=-1, keepdims=True)
    return 1.0 / se


def _dot(a, b):
    return jax.lax.dot_general(
        a, b, (((a.ndim - 1,), (0,)), ((), ())),
        preferred_element_type=jnp.float32,
        precision=jax.lax.Precision.DEFAULT)


def _vlm_sq(x, encW, encb, decW, decb):
    comp = jax.nn.relu(_dot(x, encW) + encb)
    mim = _dot(comp, decW) + decb
    return jnp.sum((mim - x) ** 2)


def _phase_a_kernel(x_ref, dcaW_ref, dcab_ref, cenW_ref, cenb_ref, cohw_ref,
                    cohb_ref, eeW_ref, eeb_ref, encW_ref, encb_ref, decW_ref,
                    decb_ref, x2_ref, proc3_ref, fl_ref, act_ref, stats_ref,
                    *, k, half):
    x = x_ref[...]
    tt = x.shape[0]
    active = jnp.ones((tt, 1), jnp.float32)
    encW = encW_ref[...]
    encb = encb_ref[...]
    decW = decW_ref[...]
    decb = decb_ref[...]

    fl = None
    for i in range(half):
        nact_i = jnp.sum(active)
        stats_ref[0, 0, 4 + i] = nact_i
        h = _dot(x, dcaW_ref[i]) + dcab_ref[i:i + 1, :]
        proc = x + _gelu(h * _kwta_mask(h, k).astype(jnp.float32))
        x = jnp.where(active > 0.0, proc, x)
        stats_ref[0, 0, 7 + i] = _vlm_sq(x, encW, encb, decW, decb)
        logits = _dot(x, eeW_ref[i]) + eeb_ref[i:i + 1, :]
        conf = _conf(logits)
        if fl is None:
            fl = logits
        else:
            fl = jnp.where(active > 0.0, logits, fl)
        active = active * (conf < _THRESHOLD).astype(jnp.float32)

    stats_ref[0, 0, 3] = jnp.sum(active)
    h = _dot(x, dcaW_ref[half]) + dcab_ref[half:half + 1, :]
    proc3 = x + _gelu(h * _kwta_mask(h, k).astype(jnp.float32))
    cohw = cohw_ref[...]
    cohb = cohb_ref[0, 0]
    for j in range(cenW_ref.shape[0]):
        sims = _gelu(_dot(proc3, cenW_ref[j]) + cenb_ref[j:j + 1, :])
        coh = jnp.sum(sims * cohw, axis=-1, keepdims=True) + cohb
        stats_ref[0, 0, j] = jnp.sum(coh * active)

    x2_ref[...] = x
    proc3_ref[...] = proc3
    fl_ref[...] = fl
    act_ref[...] = active


def _phase_b_kernel(best_ref, x2_ref, proc3_ref, act_ref, flin_ref, dcaW_ref,
                    dcab_ref, cenW_ref, cenb_ref, eeW_ref, eeb_ref, encW_ref,
                    encb_ref, decW_ref, decb_ref, fl_ref, stats_ref,
                    *, k, n_layers, half):
    x2 = x2_ref[...]
    proc3 = proc3_ref[...]
    active = act_ref[...]
    encW = encW_ref[...]
    encb = encb_ref[...]
    decW = decW_ref[...]
    decb = decb_ref[...]
    best = best_ref[0]

    sims = _gelu(_dot(proc3, cenW_ref[best]) +
                 cenb_ref[pl.ds(best, 1), :])
    x = jnp.where(active > 0.0, sims + proc3, x2)
    stats_ref[0, 0, 0] = _vlm_sq(x, encW, encb, decW, decb)
    logits = _dot(x, eeW_ref[0]) + eeb_ref[0:1, :]
    conf = _conf(logits)
    fl = jnp.where(active > 0.0, logits, flin_ref[...])
    active = active * (conf < _THRESHOLD).astype(jnp.float32)

    for i in range(half + 1, n_layers):
        li = i - half - 1
        stats_ref[0, 0, 3 + li] = jnp.sum(active)
        h = _dot(x, dcaW_ref[li]) + dcab_ref[li:li + 1, :]
        proc = x + _gelu(h * _kwta_mask(h, k).astype(jnp.float32))
        x = jnp.where(active > 0.0, proc, x)
        stats_ref[0, 0, 1 + li] = _vlm_sq(x, encW, encb, decW, decb)
        logits = _dot(x, eeW_ref[i - half]) + eeb_ref[i - half:i - half + 1, :]
        conf = _conf(logits)
        fl = jnp.where(active > 0.0, logits, fl)
        active = active * (conf < _THRESHOLD).astype(jnp.float32)

    fl_ref[...] = fl


def _const_spec(shape):
    nd = len(shape)
    return pl.BlockSpec(shape, lambda t: (0,) * nd)


def kernel(x, dca_W, dca_b, cen_W, cen_b, coh_w, coh_b, ee_W, ee_b,
           vlm_enc_W, vlm_enc_b, vlm_dec_W, vlm_dec_b):
    b, s, d = x.shape
    n_layers = dca_W.shape[0]
    half = n_layers // 2
    n_classes = ee_W.shape[-1]
    n = b * s
    k = max(1, int(d * (1.0 - _SPARSITY)))
    tt = _TILE
    g = n // tt

    xf = x.reshape(n, d)
    cohw2 = coh_w.reshape(1, d)
    cohb2 = coh_b.reshape(1, 1)
    encb2 = vlm_enc_b.reshape(1, -1)
    decb2 = vlm_dec_b.reshape(1, -1)
    student = vlm_enc_W.shape[-1]
    nb = cen_W.shape[0]

    tok = lambda t: (t, 0)
    cparams = pltpu.CompilerParams(
        dimension_semantics=("arbitrary",),
        vmem_limit_bytes=56 * 1024 * 1024,
    )

    x2, proc3, fl_a, act, stats_a = pl.pallas_call(
        functools.partial(_phase_a_kernel, k=k, half=half),
        grid=(g,),
        in_specs=[
            pl.BlockSpec((tt, d), tok),
            _const_spec((half + 1, d, d)),
            _const_spec((half + 1, d)),
            _const_spec((nb, d, d)),
            _const_spec((nb, d)),
            _const_spec((1, d)),
            pl.BlockSpec(memory_space=pltpu.SMEM),
            _const_spec((half, d, n_classes)),
            _const_spec((half, n_classes)),
            _const_spec((d, student)),
            _const_spec((1, student)),
            _const_spec((student, d)),
            _const_spec((1, d)),
        ],
        out_specs=[
            pl.BlockSpec((tt, d), tok),
            pl.BlockSpec((tt, d), tok),
            pl.BlockSpec((tt, n_classes), tok),
            pl.BlockSpec((tt, 1), tok),
            pl.BlockSpec((1, 1, 16), lambda t: (t, 0, 0), memory_space=pltpu.SMEM),
        ],
        out_shape=[
            jax.ShapeDtypeStruct((n, d), jnp.float32),
            jax.ShapeDtypeStruct((n, d), jnp.float32),
            jax.ShapeDtypeStruct((n, n_classes), jnp.float32),
            jax.ShapeDtypeStruct((n, 1), jnp.float32),
            jax.ShapeDtypeStruct((g, 1, 16), jnp.float32),
        ],
        compiler_params=cparams,
    )(xf, dca_W[:half + 1], dca_b[:half + 1], cen_W, cen_b, cohw2, cohb2,
      ee_W[:half], ee_b[:half], vlm_enc_W, encb2, vlm_dec_W, decb2)

    nact3 = jnp.sum(stats_a[:, 0, 3])
    denom = jnp.maximum(nact3, 1.0)
    scores = jnp.sum(stats_a[:, 0, :nb], axis=0) / denom
    best = jnp.argmax(scores).astype(jnp.int32).reshape(1)

    fl, stats_b = pl.pallas_call(
        functools.partial(_phase_b_kernel, k=k, n_layers=n_layers, half=half),
        grid=(g,),
        in_specs=[
            pl.BlockSpec(memory_space=pltpu.SMEM),
            pl.BlockSpec((tt, d), tok),
            pl.BlockSpec((tt, d), tok),
            pl.BlockSpec((tt, 1), tok),
            pl.BlockSpec((tt, n_classes), tok),
            _const_spec((n_layers - half - 1, d, d)),
            _const_spec((n_layers - half - 1, d)),
            _const_spec((nb, d, d)),
            _const_spec((nb, d)),
            _const_spec((n_layers - half, d, n_classes)),
            _const_spec((n_layers - half, n_classes)),
            _const_spec((d, student)),
            _const_spec((1, student)),
            _const_spec((student, d)),
            _const_spec((1, d)),
        ],
        out_specs=[
            pl.BlockSpec((tt, n_classes), tok),
            pl.BlockSpec((1, 1, 16), lambda t: (t, 0, 0), memory_space=pltpu.SMEM),
        ],
        out_shape=[
            jax.ShapeDtypeStruct((n, n_classes), jnp.float32),
            jax.ShapeDtypeStruct((g, 1, 16), jnp.float32),
        ],
        input_output_aliases={4: 0},
        compiler_params=cparams,
    )(best, x2, proc3, act, fl_a, dca_W[half + 1:], dca_b[half + 1:],
      cen_W, cen_b, ee_W[half:], ee_b[half:], vlm_enc_W, encb2, vlm_dec_W,
      decb2)

    nact = jnp.stack([jnp.sum(stats_a[:, 0, 4]), jnp.sum(stats_a[:, 0, 5]),
                      jnp.sum(stats_a[:, 0, 6]), nact3,
                      jnp.sum(stats_b[:, 0, 3]), jnp.sum(stats_b[:, 0, 4])])
    sq = jnp.stack([jnp.sum(stats_a[:, 0, 7]), jnp.sum(stats_a[:, 0, 8]),
                    jnp.sum(stats_a[:, 0, 9]), jnp.sum(stats_b[:, 0, 0]),
                    jnp.sum(stats_b[:, 0, 1]), jnp.sum(stats_b[:, 0, 2])])
    any_act = (nact > 0.0).astype(jnp.float32)
    vloss = sq / jnp.float32(n * d)
    loss_sum = jnp.sum(vloss * any_act)
    cnt = jnp.sum(any_act)
    avg_layers = jnp.sum(nact) / jnp.float32(n)
    avg_vloss = loss_sum / jnp.maximum(cnt, 1.0)
    return fl.reshape(b, s, n_classes), avg_layers, avg_vloss


# speedup vs baseline: 4.9998x; 1.7320x over previous
import functools

import jax
import jax.numpy as jnp
from jax.experimental import pallas as pl
from jax.experimental.pallas import tpu as pltpu

_SPARSITY = 0.8
_THRESHOLD = 0.85
_BISECT_ITERS = 22
_TILE = 512
_INV_SQRT2 = 0.7071067811865476


def _gelu(v):
    return 0.5 * v * (1.0 + jax.lax.erf(v * _INV_SQRT2))


def _dot(a, b):
    return jax.lax.dot_general(
        a, b, (((a.ndim - 1,), (0,)), ((), ())),
        preferred_element_type=jnp.float32,
        precision=jax.lax.Precision.DEFAULT)


def _kwta_mask_t(ht, k):
    ah = jnp.abs(ht)
    mx = jnp.max(ah, axis=0, keepdims=True)
    lo = jnp.zeros_like(mx)
    hi = mx * (1.0 + 2.0 ** -12) + 1e-30
    kf = jnp.float32(k)
    for _ in range(_BISECT_ITERS):
        mid = 0.5 * (lo + hi)
        cnt = jnp.sum((ah >= mid).astype(jnp.float32), axis=0, keepdims=True)
        pred = cnt >= kf
        lo = jnp.where(pred, mid, lo)
        hi = jnp.where(pred, hi, mid)
    return ah >= lo


def _conf_t(logits_t):
    m = jnp.max(logits_t, axis=0, keepdims=True)
    se = jnp.sum(jnp.exp(logits_t - m), axis=0, keepdims=True)
    return 1.0 / se


def _vlm_sq_t(xt, encWT, encbT, decWT, decbT):
    comp = jax.nn.relu(_dot(encWT, xt) + encbT)
    mim = _dot(decWT, comp) + decbT
    return jnp.sum((mim - xt) ** 2)


def _dca_t(xt, wt, bt, active, k):
    ht = _dot(wt, xt) + bt
    proc = xt + _gelu(ht * _kwta_mask_t(ht, k).astype(jnp.float32))
    return jnp.where(active > 0.0, proc, xt)


def _phase_a_kernel(x_ref, dcaWT_ref, dcabT_ref, cenWT_ref, cenbT_ref,
                    cohwT_ref, cohb_ref, eeWT_ref, eebT_ref, encWT_ref,
                    encbT_ref, decWT_ref, decbT_ref, x2_ref, proc3_ref,
                    flt_ref, act_ref, stats_ref, *, k, half):
    xt = x_ref[...].T
    tt = xt.shape[1]
    active = jnp.ones((1, tt), jnp.float32)
    encWT = encWT_ref[...]
    encbT = encbT_ref[...]
    decWT = decWT_ref[...]
    decbT = decbT_ref[...]

    for i in range(half):
        stats_ref[0, 0, 4 + i] = jnp.sum(active)
        xt = _dca_t(xt, dcaWT_ref[i], dcabT_ref[i], active, k)
        stats_ref[0, 0, 7 + i] = _vlm_sq_t(xt, encWT, encbT, decWT, decbT)
        logits_t = _dot(eeWT_ref[i], xt) + eebT_ref[i]
        conf = _conf_t(logits_t)
        if i == 0:
            flt_ref[...] = logits_t
        else:
            flt_ref[...] = jnp.where(active > 0.0, logits_t, flt_ref[...])
        active = active * (conf < _THRESHOLD).astype(jnp.float32)

    stats_ref[0, 0, 3] = jnp.sum(active)
    ht = _dot(dcaWT_ref[half], xt) + dcabT_ref[half]
    proc3 = xt + _gelu(ht * _kwta_mask_t(ht, k).astype(jnp.float32))
    cohwT = cohwT_ref[...]
    cohb = cohb_ref[0, 0]
    for j in range(cenWT_ref.shape[0]):
        sims = _gelu(_dot(cenWT_ref[j], proc3) + cenbT_ref[j])
        coh = jnp.sum(sims * cohwT, axis=0, keepdims=True) + cohb
        stats_ref[0, 0, j] = jnp.sum(coh * active)

    x2_ref[...] = xt
    proc3_ref[...] = proc3
    act_ref[...] = active.reshape(1, 1, tt)


def _phase_b_kernel(best_ref, x2_ref, proc3_ref, act_ref, flt_ref, dcaWT_ref,
                    dcabT_ref, cenWT_ref, cenbT_ref, eeWT_ref, eebT_ref,
                    encWT_ref, encbT_ref, decWT_ref, decbT_ref, fl_ref,
                    stats_ref, *, k, n_layers, half):
    x2 = x2_ref[...]
    proc3 = proc3_ref[...]
    tt = x2.shape[1]
    active = act_ref[0]
    encWT = encWT_ref[...]
    encbT = encbT_ref[...]
    decWT = decWT_ref[...]
    decbT = decbT_ref[...]
    best = best_ref[0]

    sims = _gelu(_dot(cenWT_ref[best], proc3) + cenbT_ref[best])
    xt = jnp.where(active > 0.0, sims + proc3, x2)
    stats_ref[0, 0, 0] = _vlm_sq_t(xt, encWT, encbT, decWT, decbT)
    logits_t = _dot(eeWT_ref[0], xt) + eebT_ref[0]
    conf = _conf_t(logits_t)
    flt = jnp.where(active > 0.0, logits_t, flt_ref[...])
    active = active * (conf < _THRESHOLD).astype(jnp.float32)

    for i in range(half + 1, n_layers):
        li = i - half - 1
        stats_ref[0, 0, 3 + li] = jnp.sum(active)
        xt = _dca_t(xt, dcaWT_ref[li], dcabT_ref[li], active, k)
        stats_ref[0, 0, 1 + li] = _vlm_sq_t(xt, encWT, encbT, decWT, decbT)
        logits_t = _dot(eeWT_ref[i - half], xt) + eebT_ref[i - half]
        conf = _conf_t(logits_t)
        flt = jnp.where(active > 0.0, logits_t, flt)
        active = active * (conf < _THRESHOLD).astype(jnp.float32)

    fl_ref[...] = flt.T


def _const_spec(shape):
    nd = len(shape)
    return pl.BlockSpec(shape, lambda t: (0,) * nd)


def kernel(x, dca_W, dca_b, cen_W, cen_b, coh_w, coh_b, ee_W, ee_b,
           vlm_enc_W, vlm_enc_b, vlm_dec_W, vlm_dec_b):
    b, s, d = x.shape
    n_layers = dca_W.shape[0]
    half = n_layers // 2
    n_classes = ee_W.shape[-1]
    n = b * s
    k = max(1, int(d * (1.0 - _SPARSITY)))
    tt = _TILE
    g = n // tt

    xf = x.reshape(n, d)
    dcaWT = jnp.swapaxes(dca_W, 1, 2)
    dcabT = dca_b[..., None]
    cenWT = jnp.swapaxes(cen_W, 1, 2)
    cenbT = cen_b[..., None]
    eeWT = jnp.swapaxes(ee_W, 1, 2)
    eebT = ee_b[..., None]
    encWT = vlm_enc_W.T
    encbT = vlm_enc_b[:, None]
    decWT = vlm_dec_W.T
    decbT = vlm_dec_b[:, None]
    cohwT = coh_w[:, None]
    cohb2 = coh_b.reshape(1, 1)
    student = vlm_enc_W.shape[-1]
    nb = cen_W.shape[0]

    tcol = lambda t: (0, t)
    cparams = pltpu.CompilerParams(
        dimension_semantics=("arbitrary",),
        vmem_limit_bytes=56 * 1024 * 1024,
    )

    x2, proc3, flt_a, act, stats_a = pl.pallas_call(
        functools.partial(_phase_a_kernel, k=k, half=half),
        grid=(g,),
        in_specs=[
            pl.BlockSpec((tt, d), lambda t: (t, 0)),
            _const_spec((half + 1, d, d)),
            _const_spec((half + 1, d, 1)),
            _const_spec((nb, d, d)),
            _const_spec((nb, d, 1)),
            _const_spec((d, 1)),
            pl.BlockSpec(memory_space=pltpu.SMEM),
            _const_spec((half, n_classes, d)),
            _const_spec((half, n_classes, 1)),
            _const_spec((student, d)),
            _const_spec((student, 1)),
            _const_spec((d, student)),
            _const_spec((d, 1)),
        ],
        out_specs=[
            pl.BlockSpec((d, tt), tcol),
            pl.BlockSpec((d, tt), tcol),
            pl.BlockSpec((n_classes, tt), tcol),
            pl.BlockSpec((1, 1, tt), lambda t: (0, 0, t)),
            pl.BlockSpec((1, 1, 16), lambda t: (t, 0, 0),
                         memory_space=pltpu.SMEM),
        ],
        out_shape=[
            jax.ShapeDtypeStruct((d, n), jnp.float32),
            jax.ShapeDtypeStruct((d, n), jnp.float32),
            jax.ShapeDtypeStruct((n_classes, n), jnp.float32),
            jax.ShapeDtypeStruct((1, 1, n), jnp.float32),
            jax.ShapeDtypeStruct((g, 1, 16), jnp.float32),
        ],
        compiler_params=cparams,
    )(xf, dcaWT[:half + 1], dcabT[:half + 1], cenWT, cenbT, cohwT, cohb2,
      eeWT[:half], eebT[:half], encWT, encbT, decWT, decbT)

    nact3 = jnp.sum(stats_a[:, 0, 3])
    denom = jnp.maximum(nact3, 1.0)
    scores = jnp.sum(stats_a[:, 0, :nb], axis=0) / denom
    best = jnp.argmax(scores).astype(jnp.int32).reshape(1)

    fl, stats_b = pl.pallas_call(
        functools.partial(_phase_b_kernel, k=k, n_layers=n_layers, half=half),
        grid=(g,),
        in_specs=[
            pl.BlockSpec(memory_space=pltpu.SMEM),
            pl.BlockSpec((d, tt), tcol),
            pl.BlockSpec((d, tt), tcol),
            pl.BlockSpec((1, 1, tt), lambda t: (0, 0, t)),
            pl.BlockSpec((n_classes, tt), tcol),
            _const_spec((n_layers - half - 1, d, d)),
            _const_spec((n_layers - half - 1, d, 1)),
            _const_spec((nb, d, d)),
            _const_spec((nb, d, 1)),
            _const_spec((n_layers - half, n_classes, d)),
            _const_spec((n_layers - half, n_classes, 1)),
            _const_spec((student, d)),
            _const_spec((student, 1)),
            _const_spec((d, student)),
            _const_spec((d, 1)),
        ],
        out_specs=[
            pl.BlockSpec((tt, n_classes), lambda t: (t, 0)),
            pl.BlockSpec((1, 1, 16), lambda t: (t, 0, 0),
                         memory_space=pltpu.SMEM),
        ],
        out_shape=[
            jax.ShapeDtypeStruct((n, n_classes), jnp.float32),
            jax.ShapeDtypeStruct((g, 1, 16), jnp.float32),
        ],
        compiler_params=cparams,
    )(best, x2, proc3, act, flt_a, dcaWT[half + 1:], dcabT[half + 1:],
      cenWT, cenbT, eeWT[half:], eebT[half:], encWT, encbT, decWT, decbT)

    nact = jnp.stack([jnp.sum(stats_a[:, 0, 4]), jnp.sum(stats_a[:, 0, 5]),
                      jnp.sum(stats_a[:, 0, 6]), nact3,
                      jnp.sum(stats_b[:, 0, 3]), jnp.sum(stats_b[:, 0, 4])])
    sq = jnp.stack([jnp.sum(stats_a[:, 0, 7]), jnp.sum(stats_a[:, 0, 8]),
                    jnp.sum(stats_a[:, 0, 9]), jnp.sum(stats_b[:, 0, 0]),
                    jnp.sum(stats_b[:, 0, 1]), jnp.sum(stats_b[:, 0, 2])])
    any_act = (nact > 0.0).astype(jnp.float32)
    vloss = sq / jnp.float32(n * d)
    loss_sum = jnp.sum(vloss * any_act)
    cnt = jnp.sum(any_act)
    avg_layers = jnp.sum(nact) / jnp.float32(n)
    avg_vloss = loss_sum / jnp.maximum(cnt, 1.0)
    return fl.reshape(b, s, n_classes), avg_layers, avg_vloss


# speedup vs baseline: 5.0373x; 1.0075x over previous
import functools

import jax
import jax.numpy as jnp
from jax.experimental import pallas as pl
from jax.experimental.pallas import tpu as pltpu

_SPARSITY = 0.8
_THRESHOLD = 0.85
_BISECT_ITERS = 20
_TILE = 512
_INV_SQRT2 = 0.7071067811865476


def _gelu(v):
    return 0.5 * v * (1.0 + jax.lax.erf(v * _INV_SQRT2))


def _dot(a, b):
    return jax.lax.dot_general(
        a, b, (((0,), (0,)), ((), ())),
        preferred_element_type=jnp.float32,
        precision=jax.lax.Precision.DEFAULT)


def _kwta_mask_t(ht, k):
    ah = jnp.abs(ht)
    mx = jnp.max(ah, axis=0, keepdims=True)
    lo = jnp.zeros_like(mx)
    hi = mx * (1.0 + 2.0 ** -12) + 1e-30
    kf = jnp.float32(k)
    for _ in range(_BISECT_ITERS):
        mid = 0.5 * (lo + hi)
        cnt = jnp.sum((ah >= mid).astype(jnp.float32), axis=0, keepdims=True)
        pred = cnt >= kf
        lo = jnp.where(pred, mid, lo)
        hi = jnp.where(pred, hi, mid)
    return ah >= lo


def _conf_t(logits_t):
    m = jnp.max(logits_t, axis=0, keepdims=True)
    se = jnp.sum(jnp.exp(logits_t - m), axis=0, keepdims=True)
    return 1.0 / se


def _vlm_sq_t(xt, encWT, encbT, decWT, decbT):
    comp = jax.nn.relu(_dot(encWT, xt) + encbT)
    mim = _dot(decWT, comp) + decbT
    return jnp.sum((mim - xt) ** 2)


def _dca_t(xt, wt, bt, active, k):
    ht = _dot(wt, xt) + bt
    proc = xt + _gelu(ht * _kwta_mask_t(ht, k).astype(jnp.float32))
    return jnp.where(active > 0.0, proc, xt)


def _phase_a_kernel(x_ref, dcaWT_ref, dcabT_ref, cenWT_ref, cenbT_ref,
                    cohwT_ref, cohb_ref, eeWT_ref, eebT_ref, encWT_ref,
                    encbT_ref, decWT_ref, decbT_ref, x2_ref, proc3_ref,
                    flt_ref, act_ref, stats_ref, *, k, half):
    xt = x_ref[...].T
    tt = xt.shape[1]
    active = jnp.ones((1, tt), jnp.float32)
    encWT = encWT_ref[...]
    encbT = encbT_ref[...]
    decWT = decWT_ref[...]
    decbT = decbT_ref[...]

    for i in range(half):
        stats_ref[0, 0, 4 + i] = jnp.sum(active)
        xt = _dca_t(xt, dcaWT_ref[i], dcabT_ref[i], active, k)
        stats_ref[0, 0, 7 + i] = _vlm_sq_t(xt, encWT, encbT, decWT, decbT)
        logits_t = _dot(eeWT_ref[i], xt) + eebT_ref[i]
        conf = _conf_t(logits_t)
        if i == 0:
            flt_ref[...] = logits_t
        else:
            flt_ref[...] = jnp.where(active > 0.0, logits_t, flt_ref[...])
        active = active * (conf < _THRESHOLD).astype(jnp.float32)

    stats_ref[0, 0, 3] = jnp.sum(active)
    ht = _dot(dcaWT_ref[half], xt) + dcabT_ref[half]
    proc3 = xt + _gelu(ht * _kwta_mask_t(ht, k).astype(jnp.float32))
    cohwT = cohwT_ref[...]
    cohb = cohb_ref[0, 0]
    for j in range(cenWT_ref.shape[0]):
        sims = _gelu(_dot(cenWT_ref[j], proc3) + cenbT_ref[j])
        coh = jnp.sum(sims * cohwT, axis=0, keepdims=True) + cohb
        stats_ref[0, 0, j] = jnp.sum(coh * active)

    x2_ref[...] = xt
    proc3_ref[...] = proc3
    act_ref[...] = active.reshape(1, 1, tt)


def _phase_b_kernel(best_ref, x2_ref, proc3_ref, act_ref, flt_ref, dcaWT_ref,
                    dcabT_ref, cenWT_ref, cenbT_ref, eeWT_ref, eebT_ref,
                    encWT_ref, encbT_ref, decWT_ref, decbT_ref, fl_ref,
                    stats_ref, *, k, n_layers, half):
    x2 = x2_ref[...]
    proc3 = proc3_ref[...]
    tt = x2.shape[1]
    active = act_ref[0]
    encWT = encWT_ref[...]
    encbT = encbT_ref[...]
    decWT = decWT_ref[...]
    decbT = decbT_ref[...]
    best = best_ref[0]

    sims = _gelu(_dot(cenWT_ref[best], proc3) + cenbT_ref[best])
    xt = jnp.where(active > 0.0, sims + proc3, x2)
    stats_ref[0, 0, 0] = _vlm_sq_t(xt, encWT, encbT, decWT, decbT)
    logits_t = _dot(eeWT_ref[0], xt) + eebT_ref[0]
    conf = _conf_t(logits_t)
    flt = jnp.where(active > 0.0, logits_t, flt_ref[...])
    active = active * (conf < _THRESHOLD).astype(jnp.float32)

    for i in range(half + 1, n_layers):
        li = i - half - 1
        stats_ref[0, 0, 3 + li] = jnp.sum(active)
        xt = _dca_t(xt, dcaWT_ref[li], dcabT_ref[li], active, k)
        stats_ref[0, 0, 1 + li] = _vlm_sq_t(xt, encWT, encbT, decWT, decbT)
        logits_t = _dot(eeWT_ref[i - half], xt) + eebT_ref[i - half]
        conf = _conf_t(logits_t)
        flt = jnp.where(active > 0.0, logits_t, flt)
        active = active * (conf < _THRESHOLD).astype(jnp.float32)

    fl_ref[...] = flt.T


def _const_spec(shape):
    nd = len(shape)
    return pl.BlockSpec(shape, lambda t: (0,) * nd)


def kernel(x, dca_W, dca_b, cen_W, cen_b, coh_w, coh_b, ee_W, ee_b,
           vlm_enc_W, vlm_enc_b, vlm_dec_W, vlm_dec_b):
    b, s, d = x.shape
    n_layers = dca_W.shape[0]
    half = n_layers // 2
    n_classes = ee_W.shape[-1]
    n = b * s
    k = max(1, int(d * (1.0 - _SPARSITY)))
    tt = _TILE
    g = n // tt

    xf = x.reshape(n, d)
    dcaWT = dca_W
    dcabT = dca_b[..., None]
    cenWT = cen_W
    cenbT = cen_b[..., None]
    eeWT = ee_W
    eebT = ee_b[..., None]
    encWT = vlm_enc_W
    encbT = vlm_enc_b[:, None]
    decWT = vlm_dec_W
    decbT = vlm_dec_b[:, None]
    cohwT = coh_w[:, None]
    cohb2 = coh_b.reshape(1, 1)
    student = vlm_enc_W.shape[-1]
    nb = cen_W.shape[0]

    tcol = lambda t: (0, t)
    cparams = pltpu.CompilerParams(
        dimension_semantics=("arbitrary",),
        vmem_limit_bytes=56 * 1024 * 1024,
    )

    x2, proc3, flt_a, act, stats_a = pl.pallas_call(
        functools.partial(_phase_a_kernel, k=k, half=half),
        grid=(g,),
        in_specs=[
            pl.BlockSpec((tt, d), lambda t: (t, 0)),
            _const_spec((half + 1, d, d)),
            _const_spec((half + 1, d, 1)),
            _const_spec((nb, d, d)),
            _const_spec((nb, d, 1)),
            _const_spec((d, 1)),
            pl.BlockSpec(memory_space=pltpu.SMEM),
            _const_spec((half, d, n_classes)),
            _const_spec((half, n_classes, 1)),
            _const_spec((d, student)),
            _const_spec((student, 1)),
            _const_spec((student, d)),
            _const_spec((d, 1)),
        ],
        out_specs=[
            pl.BlockSpec((d, tt), tcol),
            pl.BlockSpec((d, tt), tcol),
            pl.BlockSpec((n_classes, tt), tcol),
            pl.BlockSpec((1, 1, tt), lambda t: (0, 0, t)),
            pl.BlockSpec((1, 1, 16), lambda t: (t, 0, 0),
                         memory_space=pltpu.SMEM),
        ],
        out_shape=[
            jax.ShapeDtypeStruct((d, n), jnp.float32),
            jax.ShapeDtypeStruct((d, n), jnp.float32),
            jax.ShapeDtypeStruct((n_classes, n), jnp.float32),
            jax.ShapeDtypeStruct((1, 1, n), jnp.float32),
            jax.ShapeDtypeStruct((g, 1, 16), jnp.float32),
        ],
        compiler_params=cparams,
    )(xf, dcaWT[:half + 1], dcabT[:half + 1], cenWT, cenbT, cohwT, cohb2,
      eeWT[:half], eebT[:half], encWT, encbT, decWT, decbT)

    nact3 = jnp.sum(stats_a[:, 0, 3])
    denom = jnp.maximum(nact3, 1.0)
    scores = jnp.sum(stats_a[:, 0, :nb], axis=0) / denom
    best = jnp.argmax(scores).astype(jnp.int32).reshape(1)

    fl, stats_b = pl.pallas_call(
        functools.partial(_phase_b_kernel, k=k, n_layers=n_layers, half=half),
        grid=(g,),
        in_specs=[
            pl.BlockSpec(memory_space=pltpu.SMEM),
            pl.BlockSpec((d, tt), tcol),
            pl.BlockSpec((d, tt), tcol),
            pl.BlockSpec((1, 1, tt), lambda t: (0, 0, t)),
            pl.BlockSpec((n_classes, tt), tcol),
            _const_spec((n_layers - half - 1, d, d)),
            _const_spec((n_layers - half - 1, d, 1)),
            _const_spec((nb, d, d)),
            _const_spec((nb, d, 1)),
            _const_spec((n_layers - half, d, n_classes)),
            _const_spec((n_layers - half, n_classes, 1)),
            _const_spec((d, student)),
            _const_spec((student, 1)),
            _const_spec((student, d)),
            _const_spec((d, 1)),
        ],
        out_specs=[
            pl.BlockSpec((tt, n_classes), lambda t: (t, 0)),
            pl.BlockSpec((1, 1, 16), lambda t: (t, 0, 0),
                         memory_space=pltpu.SMEM),
        ],
        out_shape=[
            jax.ShapeDtypeStruct((n, n_classes), jnp.float32),
            jax.ShapeDtypeStruct((g, 1, 16), jnp.float32),
        ],
        compiler_params=cparams,
    )(best, x2, proc3, act, flt_a, dcaWT[half + 1:], dcabT[half + 1:],
      cenWT, cenbT, eeWT[half:], eebT[half:], encWT, encbT, decWT, decbT)

    nact = jnp.stack([jnp.sum(stats_a[:, 0, 4]), jnp.sum(stats_a[:, 0, 5]),
                      jnp.sum(stats_a[:, 0, 6]), nact3,
                      jnp.sum(stats_b[:, 0, 3]), jnp.sum(stats_b[:, 0, 4])])
    sq = jnp.stack([jnp.sum(stats_a[:, 0, 7]), jnp.sum(stats_a[:, 0, 8]),
                    jnp.sum(stats_a[:, 0, 9]), jnp.sum(stats_b[:, 0, 0]),
                    jnp.sum(stats_b[:, 0, 1]), jnp.sum(stats_b[:, 0, 2])])
    any_act = (nact > 0.0).astype(jnp.float32)
    vloss = sq / jnp.float32(n * d)
    loss_sum = jnp.sum(vloss * any_act)
    cnt = jnp.sum(any_act)
    avg_layers = jnp.sum(nact) / jnp.float32(n)
    avg_vloss = loss_sum / jnp.maximum(cnt, 1.0)
    return fl.reshape(b, s, n_classes), avg_layers, avg_vloss


# speedup vs baseline: 5.5792x; 1.1076x over previous
import functools

import jax
import jax.numpy as jnp
from jax.experimental import pallas as pl
from jax.experimental.pallas import tpu as pltpu

_SPARSITY = 0.8
_THRESHOLD = 0.85
_BISECT_ITERS = 20
_TILE = 512
_INV_SQRT2 = 0.7071067811865476


def _gelu(v):
    return 0.5 * v * (1.0 + jax.lax.erf(v * _INV_SQRT2))


def _dot(a, b):
    return jax.lax.dot_general(
        a, b, (((0,), (0,)), ((), ())),
        preferred_element_type=jnp.float32,
        precision=jax.lax.Precision.DEFAULT)


def _dot_std(a, b):
    return jax.lax.dot_general(
        a, b, (((1,), (0,)), ((), ())),
        preferred_element_type=jnp.float32,
        precision=jax.lax.Precision.DEFAULT)


def _kwta_mask_t(ht, k):
    ah = jnp.abs(ht)
    mx = jnp.max(ah, axis=0, keepdims=True)
    lo = jnp.zeros_like(mx)
    hi = mx * (1.0 + 2.0 ** -12) + 1e-30
    kf = jnp.float32(k)
    for _ in range(_BISECT_ITERS):
        mid = 0.5 * (lo + hi)
        cnt = jnp.sum((ah >= mid).astype(jnp.float32), axis=0, keepdims=True)
        pred = cnt >= kf
        lo = jnp.where(pred, mid, lo)
        hi = jnp.where(pred, hi, mid)
    return ah >= lo


def _conf_t(logits_t):
    m = jnp.max(logits_t, axis=0, keepdims=True)
    se = jnp.sum(jnp.exp(logits_t - m), axis=0, keepdims=True)
    return 1.0 / se


def _vlm_sq_t(xt, encWT, encbT, decWT, decbT):
    comp = jax.nn.relu(_dot_std(encWT, xt) + encbT)
    mim = _dot(decWT, comp) + decbT
    return jnp.sum((mim - xt) ** 2)


def _dca_t(xt, wt, bt, active, k):
    ht = _dot(wt, xt) + bt
    proc = xt + _gelu(ht * _kwta_mask_t(ht, k).astype(jnp.float32))
    return jnp.where(active > 0.0, proc, xt)


def _phase_a_kernel(x_ref, dcaWT_ref, dcabT_ref, cenWT_ref, cenbT_ref,
                    cohwT_ref, cohb_ref, eeWT_ref, eebT_ref, encWT_ref,
                    encbT_ref, decWT_ref, decbT_ref, x2_ref, proc3_ref,
                    flt_ref, act_ref, stats_ref, *, k, half):
    xt = x_ref[...].T
    tt = xt.shape[1]
    active = jnp.ones((1, tt), jnp.float32)
    encWT = encWT_ref[...]
    encbT = encbT_ref[...]
    decWT = decWT_ref[...]
    decbT = decbT_ref[...]

    for i in range(half):
        stats_ref[0, 0, 4 + i] = jnp.sum(active)
        xt = _dca_t(xt, dcaWT_ref[i], dcabT_ref[i], active, k)
        stats_ref[0, 0, 7 + i] = _vlm_sq_t(xt, encWT, encbT, decWT, decbT)
        logits_t = _dot_std(eeWT_ref[i], xt) + eebT_ref[i]
        conf = _conf_t(logits_t)
        if i == 0:
            flt_ref[...] = logits_t
        else:
            flt_ref[...] = jnp.where(active > 0.0, logits_t, flt_ref[...])
        active = active * (conf < _THRESHOLD).astype(jnp.float32)

    stats_ref[0, 0, 3] = jnp.sum(active)
    ht = _dot(dcaWT_ref[half], xt) + dcabT_ref[half]
    proc3 = xt + _gelu(ht * _kwta_mask_t(ht, k).astype(jnp.float32))
    cohwT = cohwT_ref[...]
    cohb = cohb_ref[0, 0]
    for j in range(cenWT_ref.shape[0]):
        sims = _gelu(_dot(cenWT_ref[j], proc3) + cenbT_ref[j])
        coh = jnp.sum(sims * cohwT, axis=0, keepdims=True) + cohb
        stats_ref[0, 0, j] = jnp.sum(coh * active)

    x2_ref[...] = xt
    proc3_ref[...] = proc3
    act_ref[...] = active.reshape(1, 1, tt)


def _phase_b_kernel(best_ref, x2_ref, proc3_ref, act_ref, flt_ref, dcaWT_ref,
                    dcabT_ref, cenWT_ref, cenbT_ref, eeWT_ref, eebT_ref,
                    encWT_ref, encbT_ref, decWT_ref, decbT_ref, fl_ref,
                    stats_ref, *, k, n_layers, half):
    x2 = x2_ref[...]
    proc3 = proc3_ref[...]
    tt = x2.shape[1]
    active = act_ref[0]
    encWT = encWT_ref[...]
    encbT = encbT_ref[...]
    decWT = decWT_ref[...]
    decbT = decbT_ref[...]
    best = best_ref[0]

    sims = _gelu(_dot(cenWT_ref[best], proc3) + cenbT_ref[best])
    xt = jnp.where(active > 0.0, sims + proc3, x2)
    stats_ref[0, 0, 0] = _vlm_sq_t(xt, encWT, encbT, decWT, decbT)
    logits_t = _dot_std(eeWT_ref[0], xt) + eebT_ref[0]
    conf = _conf_t(logits_t)
    flt = jnp.where(active > 0.0, logits_t, flt_ref[...])
    active = active * (conf < _THRESHOLD).astype(jnp.float32)

    for i in range(half + 1, n_layers):
        li = i - half - 1
        stats_ref[0, 0, 3 + li] = jnp.sum(active)
        xt = _dca_t(xt, dcaWT_ref[li], dcabT_ref[li], active, k)
        stats_ref[0, 0, 1 + li] = _vlm_sq_t(xt, encWT, encbT, decWT, decbT)
        logits_t = _dot_std(eeWT_ref[i - half], xt) + eebT_ref[i - half]
        conf = _conf_t(logits_t)
        flt = jnp.where(active > 0.0, logits_t, flt)
        active = active * (conf < _THRESHOLD).astype(jnp.float32)

    fl_ref[0] = flt


def _const_spec(shape):
    nd = len(shape)
    return pl.BlockSpec(shape, lambda t: (0,) * nd)


def kernel(x, dca_W, dca_b, cen_W, cen_b, coh_w, coh_b, ee_W, ee_b,
           vlm_enc_W, vlm_enc_b, vlm_dec_W, vlm_dec_b):
    b, s, d = x.shape
    n_layers = dca_W.shape[0]
    half = n_layers // 2
    n_classes = ee_W.shape[-1]
    n = b * s
    k = max(1, int(d * (1.0 - _SPARSITY)))
    tt = _TILE
    g = n // tt

    xf = x.reshape(n, d)
    dcaWT = dca_W
    dcabT = dca_b[..., None]
    cenWT = cen_W
    cenbT = cen_b[..., None]
    eeWT = jnp.swapaxes(ee_W, 1, 2)
    eebT = ee_b[..., None]
    encWT = vlm_enc_W.T
    encbT = vlm_enc_b[:, None]
    decWT = vlm_dec_W
    decbT = vlm_dec_b[:, None]
    cohwT = coh_w[:, None]
    cohb2 = coh_b.reshape(1, 1)
    student = vlm_enc_W.shape[-1]
    nb = cen_W.shape[0]

    tcol = lambda t: (0, t)
    cparams = pltpu.CompilerParams(
        dimension_semantics=("arbitrary",),
        vmem_limit_bytes=56 * 1024 * 1024,
    )

    x2, proc3, flt_a, act, stats_a = pl.pallas_call(
        functools.partial(_phase_a_kernel, k=k, half=half),
        grid=(g,),
        in_specs=[
            pl.BlockSpec((tt, d), lambda t: (t, 0)),
            _const_spec((half + 1, d, d)),
            _const_spec((half + 1, d, 1)),
            _const_spec((nb, d, d)),
            _const_spec((nb, d, 1)),
            _const_spec((d, 1)),
            pl.BlockSpec(memory_space=pltpu.SMEM),
            _const_spec((half, n_classes, d)),
            _const_spec((half, n_classes, 1)),
            _const_spec((student, d)),
            _const_spec((student, 1)),
            _const_spec((student, d)),
            _const_spec((d, 1)),
        ],
        out_specs=[
            pl.BlockSpec((d, tt), tcol),
            pl.BlockSpec((d, tt), tcol),
            pl.BlockSpec((n_classes, tt), tcol),
            pl.BlockSpec((1, 1, tt), lambda t: (0, 0, t)),
            pl.BlockSpec((1, 1, 16), lambda t: (t, 0, 0),
                         memory_space=pltpu.SMEM),
        ],
        out_shape=[
            jax.ShapeDtypeStruct((d, n), jnp.float32),
            jax.ShapeDtypeStruct((d, n), jnp.float32),
            jax.ShapeDtypeStruct((n_classes, n), jnp.float32),
            jax.ShapeDtypeStruct((1, 1, n), jnp.float32),
            jax.ShapeDtypeStruct((g, 1, 16), jnp.float32),
        ],
        compiler_params=cparams,
    )(xf, dcaWT[:half + 1], dcabT[:half + 1], cenWT, cenbT, cohwT, cohb2,
      eeWT[:half], eebT[:half], encWT, encbT, decWT, decbT)

    nact3 = jnp.sum(stats_a[:, 0, 3])
    denom = jnp.maximum(nact3, 1.0)
    scores = jnp.sum(stats_a[:, 0, :nb], axis=0) / denom
    best = jnp.argmax(scores).astype(jnp.int32).reshape(1)

    fl, stats_b = pl.pallas_call(
        functools.partial(_phase_b_kernel, k=k, n_layers=n_layers, half=half),
        grid=(g,),
        in_specs=[
            pl.BlockSpec(memory_space=pltpu.SMEM),
            pl.BlockSpec((d, tt), tcol),
            pl.BlockSpec((d, tt), tcol),
            pl.BlockSpec((1, 1, tt), lambda t: (0, 0, t)),
            pl.BlockSpec((n_classes, tt), tcol),
            _const_spec((n_layers - half - 1, d, d)),
            _const_spec((n_layers - half - 1, d, 1)),
            _const_spec((nb, d, d)),
            _const_spec((nb, d, 1)),
            _const_spec((n_layers - half, n_classes, d)),
            _const_spec((n_layers - half, n_classes, 1)),
            _const_spec((student, d)),
            _const_spec((student, 1)),
            _const_spec((student, d)),
            _const_spec((d, 1)),
        ],
        out_specs=[
            pl.BlockSpec((1, n_classes, tt),
                         lambda t, _spt=s // tt: (t // _spt, 0, t % _spt)),
            pl.BlockSpec((1, 1, 16), lambda t: (t, 0, 0),
                         memory_space=pltpu.SMEM),
        ],
        out_shape=[
            jax.ShapeDtypeStruct((b, n_classes, s), jnp.float32),
            jax.ShapeDtypeStruct((g, 1, 16), jnp.float32),
        ],
        compiler_params=cparams,
    )(best, x2, proc3, act, flt_a, dcaWT[half + 1:], dcabT[half + 1:],
      cenWT, cenbT, eeWT[half:], eebT[half:], encWT, encbT, decWT, decbT)

    nact = jnp.stack([jnp.sum(stats_a[:, 0, 4]), jnp.sum(stats_a[:, 0, 5]),
                      jnp.sum(stats_a[:, 0, 6]), nact3,
                      jnp.sum(stats_b[:, 0, 3]), jnp.sum(stats_b[:, 0, 4])])
    sq = jnp.stack([jnp.sum(stats_a[:, 0, 7]), jnp.sum(stats_a[:, 0, 8]),
                    jnp.sum(stats_a[:, 0, 9]), jnp.sum(stats_b[:, 0, 0]),
                    jnp.sum(stats_b[:, 0, 1]), jnp.sum(stats_b[:, 0, 2])])
    any_act = (nact > 0.0).astype(jnp.float32)
    vloss = sq / jnp.float32(n * d)
    loss_sum = jnp.sum(vloss * any_act)
    cnt = jnp.sum(any_act)
    avg_layers = jnp.sum(nact) / jnp.float32(n)
    avg_vloss = loss_sum / jnp.maximum(cnt, 1.0)
    return jnp.transpose(fl, (0, 2, 1)), avg_layers, avg_vloss


# speedup vs baseline: 6.3339x; 1.1353x over previous
import functools

import jax
import jax.numpy as jnp
from jax.experimental import pallas as pl
from jax.experimental.pallas import tpu as pltpu

_SPARSITY = 0.8
_THRESHOLD = 0.85
_BISECT_ITERS = 20
_TILE = 1024
_INV_SQRT2 = 0.7071067811865476


def _gelu(v):
    return 0.5 * v * (1.0 + jax.lax.erf(v * _INV_SQRT2))


def _dot(a, b):
    return jax.lax.dot_general(
        a, b, (((0,), (0,)), ((), ())),
        preferred_element_type=jnp.float32,
        precision=jax.lax.Precision.DEFAULT)


def _dot_std(a, b):
    return jax.lax.dot_general(
        a, b, (((1,), (0,)), ((), ())),
        preferred_element_type=jnp.float32,
        precision=jax.lax.Precision.DEFAULT)


def _kwta_mask_t(ht, k):
    ah = jnp.abs(ht)
    mx = jnp.max(ah, axis=0, keepdims=True)
    lo = jnp.zeros_like(mx)
    hi = mx * (1.0 + 2.0 ** -12) + 1e-30
    kf = jnp.float32(k)
    for _ in range(_BISECT_ITERS):
        mid = 0.5 * (lo + hi)
        cnt = jnp.sum((ah >= mid).astype(jnp.float32), axis=0, keepdims=True)
        pred = cnt >= kf
        lo = jnp.where(pred, mid, lo)
        hi = jnp.where(pred, hi, mid)
    return ah >= lo


def _conf_t(logits_t):
    m = jnp.max(logits_t, axis=0, keepdims=True)
    se = jnp.sum(jnp.exp(logits_t - m), axis=0, keepdims=True)
    return 1.0 / se


def _vlm_sq_t(xt, encWT, encbT, decWT, decbT):
    comp = jax.nn.relu(_dot_std(encWT, xt) + encbT)
    mim = _dot(decWT, comp) + decbT
    return jnp.sum((mim - xt) ** 2)


def _dca_t(xt, wt, bt, active, k):
    ht = _dot(wt, xt) + bt
    proc = xt + _gelu(ht * _kwta_mask_t(ht, k).astype(jnp.float32))
    return jnp.where(active > 0.0, proc, xt)


def _phase_a_kernel(x_ref, dcaWT_ref, dcabT_ref, cenWT_ref, cenbT_ref,
                    cohwT_ref, cohb_ref, eeWT_ref, eebT_ref, encWT_ref,
                    encbT_ref, decWT_ref, decbT_ref, x2_ref, proc3_ref,
                    flt_ref, act_ref, stats_ref, *, k, half):
    xt = x_ref[...].T
    tt = xt.shape[1]
    active = jnp.ones((1, tt), jnp.float32)
    encWT = encWT_ref[...]
    encbT = encbT_ref[...]
    decWT = decWT_ref[...]
    decbT = decbT_ref[...]

    for i in range(half):
        stats_ref[0, 0, 4 + i] = jnp.sum(active)
        xt = _dca_t(xt, dcaWT_ref[i], dcabT_ref[i], active, k)
        stats_ref[0, 0, 7 + i] = _vlm_sq_t(xt, encWT, encbT, decWT, decbT)
        logits_t = _dot_std(eeWT_ref[i], xt) + eebT_ref[i]
        conf = _conf_t(logits_t)
        if i == 0:
            flt_ref[...] = logits_t
        else:
            flt_ref[...] = jnp.where(active > 0.0, logits_t, flt_ref[...])
        active = active * (conf < _THRESHOLD).astype(jnp.float32)

    stats_ref[0, 0, 3] = jnp.sum(active)
    ht = _dot(dcaWT_ref[half], xt) + dcabT_ref[half]
    proc3 = xt + _gelu(ht * _kwta_mask_t(ht, k).astype(jnp.float32))
    cohwT = cohwT_ref[...]
    cohb = cohb_ref[0, 0]
    for j in range(cenWT_ref.shape[0]):
        sims = _gelu(_dot(cenWT_ref[j], proc3) + cenbT_ref[j])
        coh = jnp.sum(sims * cohwT, axis=0, keepdims=True) + cohb
        stats_ref[0, 0, j] = jnp.sum(coh * active)

    x2_ref[...] = xt
    proc3_ref[...] = proc3
    act_ref[...] = active.reshape(1, 1, tt)


def _phase_b_kernel(best_ref, x2_ref, proc3_ref, act_ref, flt_ref, dcaWT_ref,
                    dcabT_ref, cenWT_ref, cenbT_ref, eeWT_ref, eebT_ref,
                    encWT_ref, encbT_ref, decWT_ref, decbT_ref, fl_ref,
                    stats_ref, *, k, n_layers, half):
    x2 = x2_ref[...]
    proc3 = proc3_ref[...]
    tt = x2.shape[1]
    active = act_ref[0]
    encWT = encWT_ref[...]
    encbT = encbT_ref[...]
    decWT = decWT_ref[...]
    decbT = decbT_ref[...]
    best = best_ref[0]

    sims = _gelu(_dot(cenWT_ref[best], proc3) + cenbT_ref[best])
    xt = jnp.where(active > 0.0, sims + proc3, x2)
    stats_ref[0, 0, 0] = _vlm_sq_t(xt, encWT, encbT, decWT, decbT)
    logits_t = _dot_std(eeWT_ref[0], xt) + eebT_ref[0]
    conf = _conf_t(logits_t)
    flt = jnp.where(active > 0.0, logits_t, flt_ref[...])
    active = active * (conf < _THRESHOLD).astype(jnp.float32)

    for i in range(half + 1, n_layers):
        li = i - half - 1
        stats_ref[0, 0, 3 + li] = jnp.sum(active)
        xt = _dca_t(xt, dcaWT_ref[li], dcabT_ref[li], active, k)
        stats_ref[0, 0, 1 + li] = _vlm_sq_t(xt, encWT, encbT, decWT, decbT)
        logits_t = _dot_std(eeWT_ref[i - half], xt) + eebT_ref[i - half]
        conf = _conf_t(logits_t)
        flt = jnp.where(active > 0.0, logits_t, flt)
        active = active * (conf < _THRESHOLD).astype(jnp.float32)

    fl_ref[0] = flt


def _const_spec(shape):
    nd = len(shape)
    return pl.BlockSpec(shape, lambda t: (0,) * nd)


def kernel(x, dca_W, dca_b, cen_W, cen_b, coh_w, coh_b, ee_W, ee_b,
           vlm_enc_W, vlm_enc_b, vlm_dec_W, vlm_dec_b):
    b, s, d = x.shape
    n_layers = dca_W.shape[0]
    half = n_layers // 2
    n_classes = ee_W.shape[-1]
    n = b * s
    k = max(1, int(d * (1.0 - _SPARSITY)))
    tt = _TILE
    g = n // tt

    xf = x.reshape(n, d)
    dcaWT = dca_W
    dcabT = dca_b[..., None]
    cenWT = cen_W
    cenbT = cen_b[..., None]
    eeWT = jnp.swapaxes(ee_W, 1, 2)
    eebT = ee_b[..., None]
    encWT = vlm_enc_W.T
    encbT = vlm_enc_b[:, None]
    decWT = vlm_dec_W
    decbT = vlm_dec_b[:, None]
    cohwT = coh_w[:, None]
    cohb2 = coh_b.reshape(1, 1)
    student = vlm_enc_W.shape[-1]
    nb = cen_W.shape[0]

    tcol = lambda t: (0, t)
    cparams = pltpu.CompilerParams(
        dimension_semantics=("arbitrary",),
        vmem_limit_bytes=56 * 1024 * 1024,
    )

    x2, proc3, flt_a, act, stats_a = pl.pallas_call(
        functools.partial(_phase_a_kernel, k=k, half=half),
        grid=(g,),
        in_specs=[
            pl.BlockSpec((tt, d), lambda t: (t, 0)),
            _const_spec((half + 1, d, d)),
            _const_spec((half + 1, d, 1)),
            _const_spec((nb, d, d)),
            _const_spec((nb, d, 1)),
            _const_spec((d, 1)),
            pl.BlockSpec(memory_space=pltpu.SMEM),
            _const_spec((half, n_classes, d)),
            _const_spec((half, n_classes, 1)),
            _const_spec((student, d)),
            _const_spec((student, 1)),
            _const_spec((student, d)),
            _const_spec((d, 1)),
        ],
        out_specs=[
            pl.BlockSpec((d, tt), tcol),
            pl.BlockSpec((d, tt), tcol),
            pl.BlockSpec((n_classes, tt), tcol),
            pl.BlockSpec((1, 1, tt), lambda t: (0, 0, t)),
            pl.BlockSpec((1, 1, 16), lambda t: (t, 0, 0),
                         memory_space=pltpu.SMEM),
        ],
        out_shape=[
            jax.ShapeDtypeStruct((d, n), jnp.float32),
            jax.ShapeDtypeStruct((d, n), jnp.float32),
            jax.ShapeDtypeStruct((n_classes, n), jnp.float32),
            jax.ShapeDtypeStruct((1, 1, n), jnp.float32),
            jax.ShapeDtypeStruct((g, 1, 16), jnp.float32),
        ],
        compiler_params=cparams,
    )(xf, dcaWT[:half + 1], dcabT[:half + 1], cenWT, cenbT, cohwT, cohb2,
      eeWT[:half], eebT[:half], encWT, encbT, decWT, decbT)

    nact3 = jnp.sum(stats_a[:, 0, 3])
    denom = jnp.maximum(nact3, 1.0)
    scores = jnp.sum(stats_a[:, 0, :nb], axis=0) / denom
    best = jnp.argmax(scores).astype(jnp.int32).reshape(1)

    fl, stats_b = pl.pallas_call(
        functools.partial(_phase_b_kernel, k=k, n_layers=n_layers, half=half),
        grid=(g,),
        in_specs=[
            pl.BlockSpec(memory_space=pltpu.SMEM),
            pl.BlockSpec((d, tt), tcol),
            pl.BlockSpec((d, tt), tcol),
            pl.BlockSpec((1, 1, tt), lambda t: (0, 0, t)),
            pl.BlockSpec((n_classes, tt), tcol),
            _const_spec((n_layers - half - 1, d, d)),
            _const_spec((n_layers - half - 1, d, 1)),
            _const_spec((nb, d, d)),
            _const_spec((nb, d, 1)),
            _const_spec((n_layers - half, n_classes, d)),
            _const_spec((n_layers - half, n_classes, 1)),
            _const_spec((student, d)),
            _const_spec((student, 1)),
            _const_spec((student, d)),
            _const_spec((d, 1)),
        ],
        out_specs=[
            pl.BlockSpec((1, n_classes, tt),
                         lambda t, _spt=s // tt: (t // _spt, 0, t % _spt)),
            pl.BlockSpec((1, 1, 16), lambda t: (t, 0, 0),
                         memory_space=pltpu.SMEM),
        ],
        out_shape=[
            jax.ShapeDtypeStruct((b, n_classes, s), jnp.float32),
            jax.ShapeDtypeStruct((g, 1, 16), jnp.float32),
        ],
        compiler_params=cparams,
    )(best, x2, proc3, act, flt_a, dcaWT[half + 1:], dcabT[half + 1:],
      cenWT, cenbT, eeWT[half:], eebT[half:], encWT, encbT, decWT, decbT)

    nact = jnp.stack([jnp.sum(stats_a[:, 0, 4]), jnp.sum(stats_a[:, 0, 5]),
                      jnp.sum(stats_a[:, 0, 6]), nact3,
                      jnp.sum(stats_b[:, 0, 3]), jnp.sum(stats_b[:, 0, 4])])
    sq = jnp.stack([jnp.sum(stats_a[:, 0, 7]), jnp.sum(stats_a[:, 0, 8]),
                    jnp.sum(stats_a[:, 0, 9]), jnp.sum(stats_b[:, 0, 0]),
                    jnp.sum(stats_b[:, 0, 1]), jnp.sum(stats_b[:, 0, 2])])
    any_act = (nact > 0.0).astype(jnp.float32)
    vloss = sq / jnp.float32(n * d)
    loss_sum = jnp.sum(vloss * any_act)
    cnt = jnp.sum(any_act)
    avg_layers = jnp.sum(nact) / jnp.float32(n)
    avg_vloss = loss_sum / jnp.maximum(cnt, 1.0)
    return jnp.transpose(fl, (0, 2, 1)), avg_layers, avg_vloss
